# baseline (speedup 1.0000x reference)
import functools

import jax
import jax.numpy as jnp
from jax.experimental import pallas as pl
from jax.experimental.pallas import tpu as pltpu

_BANK_ROWS = 65536
_BATCH = 4096
_DIM = 128
_CHUNK = 2048
_NCHUNK = _BANK_ROWS // _CHUNK


def _body(output_hbm, bank_hbm, oc_hbm, bc_hbm, nb_hbm, obuf, bbuf,
          osem, bsem, wsem):
    rd_out = pltpu.make_async_copy(output_hbm, obuf, osem)
    rd_out.start()
    reads = []
    for i in range(_NCHUNK):
        c = pltpu.make_async_copy(
            bank_hbm.at[pl.ds(i * _CHUNK, _CHUNK)], bbuf.at[i], bsem.at[i]
        )
        c.start()
        reads.append(c)

    writes = []
    rd_out.wait()
    for dst in (oc_hbm, nb_hbm.at[pl.ds(0, _BATCH)]):
        w = pltpu.make_async_copy(obuf, dst, wsem)
        w.start()
        writes.append(w)
    for i in range(_NCHUNK):
        reads[i].wait()
        dsts = [bc_hbm.at[pl.ds(i * _CHUNK, _CHUNK)]]
        if i > 0:
            dsts.append(nb_hbm.at[pl.ds(i * _CHUNK, _CHUNK)])
        for dst in dsts:
            w = pltpu.make_async_copy(bbuf.at[i], dst, wsem)
            w.start()
            writes.append(w)
    for w in writes:
        w.wait()


@functools.cache
def _bank_update_kernel():
    return pl.pallas_call(
        _body,
        in_specs=[
            pl.BlockSpec(memory_space=pl.ANY),
            pl.BlockSpec(memory_space=pl.ANY),
        ],
        out_specs=[
            pl.BlockSpec(memory_space=pl.ANY),
            pl.BlockSpec(memory_space=pl.ANY),
            pl.BlockSpec(memory_space=pl.ANY),
        ],
        out_shape=[
            jax.ShapeDtypeStruct((_BATCH, _DIM), jnp.float32),
            jax.ShapeDtypeStruct((_BANK_ROWS, _DIM), jnp.float32),
            jax.ShapeDtypeStruct((_BANK_ROWS, _DIM), jnp.float32),
        ],
        scratch_shapes=[
            pltpu.VMEM((_BATCH, _DIM), jnp.float32),
            pltpu.VMEM((_NCHUNK, _CHUNK, _DIM), jnp.float32),
            pltpu.SemaphoreType.DMA,
            pltpu.SemaphoreType.DMA((_NCHUNK,)),
            pltpu.SemaphoreType.DMA,
        ],
    )


def kernel(output, bank):
    out_clone, bank_clone, new_bank = _bank_update_kernel()(output, bank)
    return (out_clone, bank_clone, new_bank)


# speedup vs baseline: 1.0151x; 1.0151x over previous
import functools

import jax
import jax.numpy as jnp
from jax.experimental import pallas as pl
from jax.experimental.pallas import tpu as pltpu

_BANK_ROWS = 65536
_BATCH = 4096
_DIM = 128
_CHUNK = 2048
_NCHUNK = _BANK_ROWS // _CHUNK


def _body(output_hbm, bank_hbm, oc_hbm, bc_hbm, nb_hbm, obuf, bbuf,
          osem, bsem, wsem):
    rd_out = pltpu.make_async_copy(output_hbm, obuf, osem)
    rd_out.start()
    reads = []
    for i in range(_NCHUNK):
        c = pltpu.make_async_copy(
            bank_hbm.at[pl.ds(i * _CHUNK, _CHUNK)], bbuf.at[i], bsem.at[i]
        )
        c.start()
        reads.append(c)

    writes = []
    rd_out.wait()
    for dst in (oc_hbm, nb_hbm.at[pl.ds(0, _BATCH)]):
        w = pltpu.make_async_copy(obuf, dst, wsem)
        w.start()
        writes.append(w)
    for i in range(_NCHUNK):
        reads[i].wait()
        dsts = [bc_hbm.at[pl.ds(i * _CHUNK, _CHUNK)]]
        if i * _CHUNK >= _BATCH:
            dsts.append(nb_hbm.at[pl.ds(i * _CHUNK, _CHUNK)])
        for dst in dsts:
            w = pltpu.make_async_copy(bbuf.at[i], dst, wsem)
            w.start()
            writes.append(w)
    for w in writes:
        w.wait()


@functools.cache
def _bank_update_kernel():
    return pl.pallas_call(
        _body,
        in_specs=[
            pl.BlockSpec(memory_space=pl.ANY),
            pl.BlockSpec(memory_space=pl.ANY),
        ],
        out_specs=[
            pl.BlockSpec(memory_space=pl.ANY),
            pl.BlockSpec(memory_space=pl.ANY),
            pl.BlockSpec(memory_space=pl.ANY),
        ],
        out_shape=[
            jax.ShapeDtypeStruct((_BATCH, _DIM), jnp.float32),
            jax.ShapeDtypeStruct((_BANK_ROWS, _DIM), jnp.float32),
            jax.ShapeDtypeStruct((_BANK_ROWS, _DIM), jnp.float32),
        ],
        scratch_shapes=[
            pltpu.VMEM((_BATCH, _DIM), jnp.float32),
            pltpu.VMEM((_NCHUNK, _CHUNK, _DIM), jnp.float32),
            pltpu.SemaphoreType.DMA,
            pltpu.SemaphoreType.DMA((_NCHUNK,)),
            pltpu.SemaphoreType.DMA,
        ],
    )


def kernel(output, bank):
    out_clone, bank_clone, new_bank = _bank_update_kernel()(output, bank)
    return (out_clone, bank_clone, new_bank)


# speedup vs baseline: 1.0268x; 1.0115x over previous
import functools

import jax
import jax.numpy as jnp
from jax.experimental import pallas as pl
from jax.experimental.pallas import tpu as pltpu

_BANK_ROWS = 65536
_BATCH = 4096
_DIM = 128
_CHUNK = 4096
_NCHUNK = _BANK_ROWS // _CHUNK


def _body(output_hbm, bank_hbm, oc_hbm, bc_hbm, nb_hbm, obuf, bbuf,
          osem, bsem, wsem):
    rd_out = pltpu.make_async_copy(output_hbm, obuf, osem)
    rd_out.start()
    reads = []
    for i in range(_NCHUNK):
        c = pltpu.make_async_copy(
            bank_hbm.at[pl.ds(i * _CHUNK, _CHUNK)], bbuf.at[i], bsem.at[i]
        )
        c.start()
        reads.append(c)

    writes = []
    rd_out.wait()
    for dst in (oc_hbm, nb_hbm.at[pl.ds(0, _BATCH)]):
        w = pltpu.make_async_copy(obuf, dst, wsem)
        w.start()
        writes.append(w)
    for i in range(_NCHUNK):
        reads[i].wait()
        dsts = [bc_hbm.at[pl.ds(i * _CHUNK, _CHUNK)]]
        if i * _CHUNK >= _BATCH:
            dsts.append(nb_hbm.at[pl.ds(i * _CHUNK, _CHUNK)])
        for dst in dsts:
            w = pltpu.make_async_copy(bbuf.at[i], dst, wsem)
            w.start()
            writes.append(w)
    for w in writes:
        w.wait()


@functools.cache
def _bank_update_kernel():
    return pl.pallas_call(
        _body,
        in_specs=[
            pl.BlockSpec(memory_space=pl.ANY),
            pl.BlockSpec(memory_space=pl.ANY),
        ],
        out_specs=[
            pl.BlockSpec(memory_space=pl.ANY),
            pl.BlockSpec(memory_space=pl.ANY),
            pl.BlockSpec(memory_space=pl.ANY),
        ],
        out_shape=[
            jax.ShapeDtypeStruct((_BATCH, _DIM), jnp.float32),
            jax.ShapeDtypeStruct((_BANK_ROWS, _DIM), jnp.float32),
            jax.ShapeDtypeStruct((_BANK_ROWS, _DIM), jnp.float32),
        ],
        scratch_shapes=[
            pltpu.VMEM((_BATCH, _DIM), jnp.float32),
            pltpu.VMEM((_NCHUNK, _CHUNK, _DIM), jnp.float32),
            pltpu.SemaphoreType.DMA,
            pltpu.SemaphoreType.DMA((_NCHUNK,)),
            pltpu.SemaphoreType.DMA,
        ],
    )


def kernel(output, bank):
    out_clone, bank_clone, new_bank = _bank_update_kernel()(output, bank)
    return (out_clone, bank_clone, new_bank)
